# baseline (speedup 1.0000x reference)
import jax
import jax.numpy as jnp
from jax import lax
from jax.experimental import pallas as pl
from jax.experimental.pallas import tpu as pltpu
from jax.experimental.pallas import tpu_sc as plsc

N = 10000
E = 160000
D = 256

NC, NS = 2, 16
NW = NC * NS

BN = 400
GA = 2 * (N // BN)

EPT = 5120
EPAD = NW * EPT
GB = 64
NB_B = EPT // GB

BE = 2048

CHUNK = 320
NPAD = NW * CHUNK
CD = 4000
NCH = E // CD
GD = 64

NEG = -3.0e38
NEG_THRESH = -1.0e37
_ILV = plsc.PackFormat.INTERLEAVED


def _uv_body(x_ref, w1_ref, b1_ref, uv_ref):
    p = pl.program_id(0) // (GA // 2)
    w1a = w1_ref[:D, :]
    w1b = w1_ref[D:, :]
    w = jnp.where(p == 0, w1a - w1b, w1b)
    b = jnp.where(p == 0, b1_ref[...], jnp.zeros_like(b1_ref[...]))
    r = jnp.dot(x_ref[...], w, preferred_element_type=jnp.float32) + b
    uv_ref[...] = r.astype(jnp.bfloat16)


def _mlp2_body(m_ref, w2_ref, b2_ref, t_ref):
    t_ref[...] = (
        jnp.dot(m_ref[...], w2_ref[...], preferred_element_type=jnp.float32)
        + b2_ref[...]
    )


def _gather_body(ids_hbm, uv_hbm, msg_hbm, ids_v, gb0, gb1, ob0, ob1,
                 sg0, sg1, sw0, sw1):
    wid = lax.axis_index("s") * NC + lax.axis_index("c")
    ibase = wid * (2 * EPT)
    ebase = wid * EPT
    pltpu.sync_copy(ids_hbm.at[pl.ds(ibase, 2 * EPT)], ids_v)

    def g_copy(jb, gb, sg):
        idx = ids_v.at[pl.ds(jb * (2 * GB), 2 * GB)]
        return pltpu.make_async_copy(uv_hbm.at[idx], gb, sg)

    def w_copy(jb, ob, sw):
        return pltpu.make_async_copy(ob, msg_hbm.at[pl.ds(ebase + jb * GB, GB)],
                                     sw)

    g_copy(0, gb0, sg0).start()
    g_copy(1, gb1, sg1).start()
    bufs = ((gb0, ob0, sg0, sw0), (gb1, ob1, sg1, sw1))

    @pl.loop(0, NB_B, step=2)
    def _pair(j):
        for b in range(2):
            jb = j + b
            gb, ob, sg, sw = bufs[b]
            g_copy(jb, gb, sg).wait()

            @pl.when(jb >= 2)
            def _():
                w_copy(jb - 2, ob, sw).wait()

            @pl.loop(0, GB)
            def _edge(jj):
                for kk in range(D // 32):
                    sl = pl.ds(kk * 16, 16)
                    u = plsc.bitcast(gb[jj, sl], jnp.bfloat16)
                    v = plsc.bitcast(gb[GB + jj, sl], jnp.bfloat16)
                    r = jnp.maximum(u + v, jnp.bfloat16(0.0))
                    ob[jj, sl] = plsc.bitcast(r, jnp.int32)

            @pl.when(jb + 2 < NB_B)
            def _():
                g_copy(jb + 2, gb, sg).start()

            w_copy(jb, ob, sw).start()

    w_copy(NB_B - 2, ob0, sw0).wait()
    w_copy(NB_B - 1, ob1, sw1).wait()


def _scatter_body(row_hbm, t_hbm, out_hbm, rb0, rb1, eid_buf, acc,
                  gr0, gr1, st0, st1, sr0, sr1, sg0, sg1, sw):
    wid = lax.axis_index("s") * NC + lax.axis_index("c")
    lo = wid * CHUNK

    @pl.loop(0, CHUNK + 1)
    def _init(r):
        for k in range(D // 16):
            acc[r, pl.ds(k * 16, 16)] = jnp.full((16,), NEG, jnp.float32)

    def r_copy(cb, rb, sr):
        return pltpu.make_async_copy(row_hbm.at[pl.ds(cb * CD, CD)], rb, sr)

    def fire(j, grb, stb, sgb):
        @pl.loop(0, GD // 16)
        def _s(gg):
            pk = eid_buf[pl.ds(j * GD + gg * 16, 16)]
            stb[pl.ds(gg * 16, 16)] = pk >> 9

        pltpu.async_copy(t_hbm.at[stb], grb, sgb)

    def wait_g(grb, stb, sgb):
        pltpu.make_async_copy(t_hbm.at[stb], grb, sgb).wait()

    def proc(j, grb):
        @pl.loop(0, GD // 16)
        def _g(gg):
            pk = eid_buf[pl.ds(j * GD + gg * 16, 16)]
            nidv = pk & 511
            nids = [nidv[i] for i in range(16)]

            @pl.loop(0, D // 16, unroll=4)
            def _k(k):
                sl = pl.ds(k * 16, 16)
                for jj in range(16):
                    r = gg * 16 + jj
                    acc[nids[jj], sl] = jnp.maximum(acc[nids[jj], sl],
                                                    grb[r, sl])

    r_copy(0, rb0, sr0).start()
    r_copy(1, rb1, sr1).start()

    @pl.loop(0, NCH, step=2)
    def _cpair(c):
        for b in range(2):
            cb = c + b
            rb = (rb0, rb1)[b]
            srb = (sr0, sr1)[b]
            r_copy(cb, rb, srb).wait()
            zero_off = jnp.zeros((16,), jnp.int32)
            eids0 = lax.iota(jnp.int32, 16) + cb * CD

            @pl.loop(0, CD // 16, init_carry=(zero_off, eids0), unroll=2)
            def _filter(g, carry):
                off_v, eids = carry
                rows = rb[pl.ds(g * 16, 16)]
                m = (rows >= lo) & (rows < lo + CHUNK)
                pos = plsc.cumsum(m.astype(jnp.int32))
                idxv = off_v + pos - 1
                packed = (eids << 9) | (rows - lo)
                plsc.store_scatter(eid_buf, [idxv], packed, mask=m)
                cnt = plsc.all_reduce_population_count(m)
                return (off_v + cnt, eids + 16)

            off_v, _ = _filter
            m_cnt = off_v[0]

            @pl.when(cb + 2 < NCH)
            def _():
                r_copy(cb + 2, rb, srb).start()

            sent = jnp.full((16,), CHUNK, jnp.int32)
            for p_ in range(GD // 16):
                eid_buf[pl.ds(m_cnt + p_ * 16, 16)] = sent
            nb = (m_cnt + (GD - 1)) >> 6

            @pl.when(nb > 0)
            def _():
                fire(0, gr0, st0, sg0)

            @pl.when(nb > 1)
            def _():
                fire(1, gr1, st1, sg1)

            @pl.loop(0, nb)
            def _batch(j):
                even = (j & 1) == 0

                @pl.when(even)
                def _():
                    wait_g(gr0, st0, sg0)
                    proc(j, gr0)

                    @pl.when(j + 2 < nb)
                    def _():
                        fire(j + 2, gr0, st0, sg0)

                @pl.when(jnp.logical_not(even))
                def _():
                    wait_g(gr1, st1, sg1)
                    proc(j, gr1)

                    @pl.when(j + 2 < nb)
                    def _():
                        fire(j + 2, gr1, st1, sg1)

    @pl.loop(0, CHUNK)
    def _fin(r):
        for k in range(D // 16):
            sl = pl.ds(k * 16, 16)
            a = acc[r, sl]
            acc[r, sl] = jnp.where(a > NEG_THRESH, a, 0.0)

    pltpu.async_copy(acc.at[pl.ds(0, CHUNK)], out_hbm.at[pl.ds(lo, CHUNK)],
                     sw).wait()


def kernel(x, edge_index, W1, b1, W2, b2):
    row = edge_index[0].astype(jnp.int32)
    col = edge_index[1].astype(jnp.int32)
    b1r = b1.reshape(1, D).astype(jnp.float32)
    b2r = b2.reshape(1, D).astype(jnp.float32)

    uv = pl.pallas_call(
        _uv_body,
        grid=(GA,),
        in_specs=[
            pl.BlockSpec((BN, D), lambda i: (lax.rem(i, GA // 2), 0)),
            pl.BlockSpec((2 * D, D), lambda i: (0, 0)),
            pl.BlockSpec((1, D), lambda i: (0, 0)),
        ],
        out_specs=pl.BlockSpec((BN, D), lambda i: (i, 0)),
        out_shape=jax.ShapeDtypeStruct((2 * N, D), jnp.bfloat16),
    )(x, W1, b1r)

    pad = jnp.zeros((EPAD - E,), jnp.int32)
    row_p = jnp.concatenate([row, pad]).reshape(-1, GB)
    col_p = jnp.concatenate([col, pad]).reshape(-1, GB) + N
    ids2 = jnp.concatenate([row_p, col_p], axis=1).reshape(-1)

    mesh = plsc.VectorSubcoreMesh(
        core_axis_name="c", subcore_axis_name="s", num_cores=NC, num_subcores=NS
    )

    uv_i32 = jax.lax.bitcast_convert_type(
        uv.reshape(2 * N, D // 2, 2), jnp.int32)

    msg = pl.kernel(
        _gather_body,
        out_type=jax.ShapeDtypeStruct((EPAD, D // 2), jnp.int32),
        mesh=mesh,
        compiler_params=pltpu.CompilerParams(needs_layout_passes=False),
        scratch_types=[
            pltpu.VMEM((2 * EPT,), jnp.int32),
            pltpu.VMEM((2 * GB, D // 2), jnp.int32),
            pltpu.VMEM((2 * GB, D // 2), jnp.int32),
            pltpu.VMEM((GB, D // 2), jnp.int32),
            pltpu.VMEM((GB, D // 2), jnp.int32),
            pltpu.SemaphoreType.DMA,
            pltpu.SemaphoreType.DMA,
            pltpu.SemaphoreType.DMA,
            pltpu.SemaphoreType.DMA,
        ],
    )(ids2, uv_i32)

    msg_bf = jax.lax.bitcast_convert_type(msg, jnp.bfloat16).reshape(EPAD, D)
    t = pl.pallas_call(
        _mlp2_body,
        grid=(EPAD // BE,),
        in_specs=[
            pl.BlockSpec((BE, D), lambda i: (i, 0)),
            pl.BlockSpec((D, D), lambda i: (0, 0)),
            pl.BlockSpec((1, D), lambda i: (0, 0)),
        ],
        out_specs=pl.BlockSpec((BE, D), lambda i: (i, 0)),
        out_shape=jax.ShapeDtypeStruct((EPAD, D), jnp.float32),
    )(msg_bf, W2, b2r)

    out_pad = pl.kernel(
        _scatter_body,
        out_type=jax.ShapeDtypeStruct((NPAD, D), jnp.float32),
        mesh=mesh,
        compiler_params=pltpu.CompilerParams(needs_layout_passes=False),
        scratch_types=[
            pltpu.VMEM((CD,), jnp.int32),
            pltpu.VMEM((CD,), jnp.int32),
            pltpu.VMEM((CD + GD,), jnp.int32),
            pltpu.VMEM((CHUNK + 1, D), jnp.float32),
            pltpu.VMEM((GD, D), jnp.float32),
            pltpu.VMEM((GD, D), jnp.float32),
            pltpu.VMEM((GD,), jnp.int32),
            pltpu.VMEM((GD,), jnp.int32),
            pltpu.SemaphoreType.DMA,
            pltpu.SemaphoreType.DMA,
            pltpu.SemaphoreType.DMA,
            pltpu.SemaphoreType.DMA,
            pltpu.SemaphoreType.DMA,
        ],
    )(row, t)

    return out_pad[:N]


# speedup vs baseline: 1.0029x; 1.0029x over previous
import jax
import jax.numpy as jnp
from jax import lax
from jax.experimental import pallas as pl
from jax.experimental.pallas import tpu as pltpu
from jax.experimental.pallas import tpu_sc as plsc

N = 10000
E = 160000
D = 256

NC, NS = 2, 16
NW = NC * NS

BN = 400
GA = 2 * (N // BN)

EPT = 5120
EPAD = NW * EPT
GB = 64
NB_B = EPT // GB

BE = 2048

CHUNK = 320
NPAD = NW * CHUNK
CD = 4000
NCH = E // CD
GD = 64

NEG = -3.0e38
NEG_THRESH = -1.0e37
_ILV = plsc.PackFormat.INTERLEAVED


def _uv_body(x_ref, w1_ref, b1_ref, uv_ref):
    p = pl.program_id(0) // (GA // 2)
    w1a = w1_ref[:D, :]
    w1b = w1_ref[D:, :]
    w = jnp.where(p == 0, w1a - w1b, w1b)
    b = jnp.where(p == 0, b1_ref[...], jnp.zeros_like(b1_ref[...]))
    r = jnp.dot(x_ref[...], w, preferred_element_type=jnp.float32) + b
    uv_ref[...] = r.astype(jnp.bfloat16)


def _mlp2_body(m_ref, w2_ref, b2_ref, t_ref):
    t_ref[...] = (
        jnp.dot(m_ref[...], w2_ref[...], preferred_element_type=jnp.float32)
        + b2_ref[...]
    )


def _gather_body(ids_hbm, uv_hbm, msg_hbm, ids_v, gb0, gb1, ob0, ob1,
                 sg0, sg1, sw0, sw1):
    wid = lax.axis_index("s") * NC + lax.axis_index("c")
    ibase = wid * (2 * EPT)
    ebase = wid * EPT
    pltpu.sync_copy(ids_hbm.at[pl.ds(ibase, 2 * EPT)], ids_v)

    def g_copy(jb, gb, sg):
        idx = ids_v.at[pl.ds(jb * (2 * GB), 2 * GB)]
        return pltpu.make_async_copy(uv_hbm.at[idx], gb, sg)

    def w_copy(jb, ob, sw):
        return pltpu.make_async_copy(ob, msg_hbm.at[pl.ds(ebase + jb * GB, GB)],
                                     sw)

    g_copy(0, gb0, sg0).start()
    g_copy(1, gb1, sg1).start()
    bufs = ((gb0, ob0, sg0, sw0), (gb1, ob1, sg1, sw1))

    @pl.loop(0, NB_B, step=2)
    def _pair(j):
        for b in range(2):
            jb = j + b
            gb, ob, sg, sw = bufs[b]
            g_copy(jb, gb, sg).wait()

            @pl.when(jb >= 2)
            def _():
                w_copy(jb - 2, ob, sw).wait()

            @pl.loop(0, GB)
            def _edge(jj):
                for kk in range(D // 32):
                    sl = pl.ds(kk * 16, 16)
                    u = plsc.bitcast(gb[jj, sl], jnp.bfloat16)
                    v = plsc.bitcast(gb[GB + jj, sl], jnp.bfloat16)
                    r = jnp.maximum(u + v, jnp.bfloat16(0.0))
                    ob[jj, sl] = plsc.bitcast(r, jnp.int32)

            @pl.when(jb + 2 < NB_B)
            def _():
                g_copy(jb + 2, gb, sg).start()

            w_copy(jb, ob, sw).start()

    w_copy(NB_B - 2, ob0, sw0).wait()
    w_copy(NB_B - 1, ob1, sw1).wait()


def _scatter_body(row_hbm, t_hbm, out_hbm, rb0, rb1, eid_buf, acc,
                  gr0, gr1, st0, st1, sr0, sr1, sg0, sg1, sw):
    wid = lax.axis_index("s") * NC + lax.axis_index("c")
    lo = wid * CHUNK

    @pl.loop(0, CHUNK + 1)
    def _init(r):
        for k in range(D // 16):
            acc[r, pl.ds(k * 16, 16)] = jnp.full((16,), NEG, jnp.float32)

    def r_copy(cb, rb, sr):
        return pltpu.make_async_copy(row_hbm.at[pl.ds(cb * CD, CD)], rb, sr)

    def fire(j, grb, stb, sgb):
        @pl.loop(0, GD // 16)
        def _s(gg):
            pk = eid_buf[pl.ds(j * GD + gg * 16, 16)]
            stb[pl.ds(gg * 16, 16)] = pk >> 9

        pltpu.async_copy(t_hbm.at[stb], grb, sgb)

    def wait_g(grb, stb, sgb):
        pltpu.make_async_copy(t_hbm.at[stb], grb, sgb).wait()

    def proc(j, grb):
        @pl.loop(0, GD // 16)
        def _g(gg):
            pk = eid_buf[pl.ds(j * GD + gg * 16, 16)]
            nidv = pk & 511
            nids = [nidv[i] for i in range(16)]

            @pl.loop(0, D // 16, unroll=4)
            def _k(k):
                sl = pl.ds(k * 16, 16)
                for jj in range(16):
                    r = gg * 16 + jj
                    acc[nids[jj], sl] = jnp.maximum(acc[nids[jj], sl],
                                                    grb[r, sl])

    r_copy(0, rb0, sr0).start()
    r_copy(1, rb1, sr1).start()

    @pl.loop(0, NCH, step=2)
    def _cpair(c):
        for b in range(2):
            cb = c + b
            rb = (rb0, rb1)[b]
            srb = (sr0, sr1)[b]
            r_copy(cb, rb, srb).wait()
            zero_off = jnp.zeros((16,), jnp.int32)
            eids0 = lax.iota(jnp.int32, 16) + cb * CD

            with jax.named_scope("d_filter"):
                @pl.loop(0, CD // 16, init_carry=(zero_off, eids0), unroll=2)
                def _filter(g, carry):
                    off_v, eids = carry
                    rows = rb[pl.ds(g * 16, 16)]
                    m = (rows >= lo) & (rows < lo + CHUNK)
                    pos = plsc.cumsum(m.astype(jnp.int32))
                    idxv = off_v + pos - 1
                    packed = (eids << 9) | (rows - lo)
                    plsc.store_scatter(eid_buf, [idxv], packed, mask=m)
                    cnt = plsc.all_reduce_population_count(m)
                    return (off_v + cnt, eids + 16)

                off_v, _ = _filter
                m_cnt = off_v[0]

            @pl.when(cb + 2 < NCH)
            def _():
                r_copy(cb + 2, rb, srb).start()

            sent = jnp.full((16,), CHUNK, jnp.int32)
            for p_ in range(GD // 16):
                eid_buf[pl.ds(m_cnt + p_ * 16, 16)] = sent
            nb = (m_cnt + (GD - 1)) >> 6

            with jax.named_scope("d_fire01"):
                @pl.when(nb > 0)
                def _():
                    fire(0, gr0, st0, sg0)

                @pl.when(nb > 1)
                def _():
                    fire(1, gr1, st1, sg1)

            with jax.named_scope("d_batch"):
                @pl.loop(0, nb)
                def _batch(j):
                    even = (j & 1) == 0

                    @pl.when(even)
                    def _():
                        wait_g(gr0, st0, sg0)
                        proc(j, gr0)

                        @pl.when(j + 2 < nb)
                        def _():
                            fire(j + 2, gr0, st0, sg0)

                    @pl.when(jnp.logical_not(even))
                    def _():
                        wait_g(gr1, st1, sg1)
                        proc(j, gr1)

                        @pl.when(j + 2 < nb)
                        def _():
                            fire(j + 2, gr1, st1, sg1)

    @pl.loop(0, CHUNK)
    def _fin(r):
        for k in range(D // 16):
            sl = pl.ds(k * 16, 16)
            a = acc[r, sl]
            acc[r, sl] = jnp.where(a > NEG_THRESH, a, 0.0)

    pltpu.async_copy(acc.at[pl.ds(0, CHUNK)], out_hbm.at[pl.ds(lo, CHUNK)],
                     sw).wait()


def kernel(x, edge_index, W1, b1, W2, b2):
    row = edge_index[0].astype(jnp.int32)
    col = edge_index[1].astype(jnp.int32)
    b1r = b1.reshape(1, D).astype(jnp.float32)
    b2r = b2.reshape(1, D).astype(jnp.float32)

    uv = pl.pallas_call(
        _uv_body,
        grid=(GA,),
        in_specs=[
            pl.BlockSpec((BN, D), lambda i: (lax.rem(i, GA // 2), 0)),
            pl.BlockSpec((2 * D, D), lambda i: (0, 0)),
            pl.BlockSpec((1, D), lambda i: (0, 0)),
        ],
        out_specs=pl.BlockSpec((BN, D), lambda i: (i, 0)),
        out_shape=jax.ShapeDtypeStruct((2 * N, D), jnp.bfloat16),
    )(x, W1, b1r)

    pad = jnp.zeros((EPAD - E,), jnp.int32)
    row_p = jnp.concatenate([row, pad]).reshape(-1, GB)
    col_p = jnp.concatenate([col, pad]).reshape(-1, GB) + N
    ids2 = jnp.concatenate([row_p, col_p], axis=1).reshape(-1)

    mesh = plsc.VectorSubcoreMesh(
        core_axis_name="c", subcore_axis_name="s", num_cores=NC, num_subcores=NS
    )

    uv_i32 = jax.lax.bitcast_convert_type(
        uv.reshape(2 * N, D // 2, 2), jnp.int32)

    msg = pl.kernel(
        _gather_body,
        out_type=jax.ShapeDtypeStruct((EPAD, D // 2), jnp.int32),
        mesh=mesh,
        compiler_params=pltpu.CompilerParams(needs_layout_passes=False),
        scratch_types=[
            pltpu.VMEM((2 * EPT,), jnp.int32),
            pltpu.VMEM((2 * GB, D // 2), jnp.int32),
            pltpu.VMEM((2 * GB, D // 2), jnp.int32),
            pltpu.VMEM((GB, D // 2), jnp.int32),
            pltpu.VMEM((GB, D // 2), jnp.int32),
            pltpu.SemaphoreType.DMA,
            pltpu.SemaphoreType.DMA,
            pltpu.SemaphoreType.DMA,
            pltpu.SemaphoreType.DMA,
        ],
    )(ids2, uv_i32)

    msg_bf = jax.lax.bitcast_convert_type(msg, jnp.bfloat16).reshape(EPAD, D)
    t = pl.pallas_call(
        _mlp2_body,
        grid=(EPAD // BE,),
        in_specs=[
            pl.BlockSpec((BE, D), lambda i: (i, 0)),
            pl.BlockSpec((D, D), lambda i: (0, 0)),
            pl.BlockSpec((1, D), lambda i: (0, 0)),
        ],
        out_specs=pl.BlockSpec((BE, D), lambda i: (i, 0)),
        out_shape=jax.ShapeDtypeStruct((EPAD, D), jnp.float32),
    )(msg_bf, W2, b2r)

    out_pad = pl.kernel(
        _scatter_body,
        out_type=jax.ShapeDtypeStruct((NPAD, D), jnp.float32),
        mesh=mesh,
        compiler_params=pltpu.CompilerParams(needs_layout_passes=False),
        scratch_types=[
            pltpu.VMEM((CD,), jnp.int32),
            pltpu.VMEM((CD,), jnp.int32),
            pltpu.VMEM((CD + GD,), jnp.int32),
            pltpu.VMEM((CHUNK + 1, D), jnp.float32),
            pltpu.VMEM((GD, D), jnp.float32),
            pltpu.VMEM((GD, D), jnp.float32),
            pltpu.VMEM((GD,), jnp.int32),
            pltpu.VMEM((GD,), jnp.int32),
            pltpu.SemaphoreType.DMA,
            pltpu.SemaphoreType.DMA,
            pltpu.SemaphoreType.DMA,
            pltpu.SemaphoreType.DMA,
            pltpu.SemaphoreType.DMA,
        ],
    )(row, t)

    return out_pad[:N]


# speedup vs baseline: 1.0038x; 1.0009x over previous
import jax
import jax.numpy as jnp
from jax import lax
from jax.experimental import pallas as pl
from jax.experimental.pallas import tpu as pltpu
from jax.experimental.pallas import tpu_sc as plsc

N = 10000
E = 160000
D = 256

NC, NS = 2, 16
NW = NC * NS

BN = 400
GA = 2 * (N // BN)

EPT = 5120
EPAD = NW * EPT
GB = 64
NB_B = EPT // GB

BE = 2048

CHUNK = 320
NPAD = NW * CHUNK
CD = 4000
NCH = E // CD
GD = 64

NEG = -3.0e38
NEG_THRESH = -1.0e37
_ILV = plsc.PackFormat.INTERLEAVED


def _uv_body(x_ref, w1_ref, b1_ref, uv_ref):
    p = pl.program_id(0) // (GA // 2)
    w1a = w1_ref[:D, :]
    w1b = w1_ref[D:, :]
    w = jnp.where(p == 0, w1a - w1b, w1b)
    b = jnp.where(p == 0, b1_ref[...], jnp.zeros_like(b1_ref[...]))
    r = jnp.dot(x_ref[...], w, preferred_element_type=jnp.float32) + b
    uv_ref[...] = r.astype(jnp.bfloat16)


def _mlp2_body(m_ref, w2_ref, b2_ref, t_ref):
    t_ref[...] = (
        jnp.dot(m_ref[...], w2_ref[...], preferred_element_type=jnp.float32)
        + b2_ref[...]
    )


def _gather_body(ids_hbm, uv_hbm, msg_hbm, ids_v, gb0, gb1, ob0, ob1,
                 sg0, sg1, sw0, sw1):
    wid = lax.axis_index("s") * NC + lax.axis_index("c")
    ibase = wid * (2 * EPT)
    ebase = wid * EPT
    pltpu.sync_copy(ids_hbm.at[pl.ds(ibase, 2 * EPT)], ids_v)

    def g_copy(jb, gb, sg):
        idx = ids_v.at[pl.ds(jb * (2 * GB), 2 * GB)]
        return pltpu.make_async_copy(uv_hbm.at[idx], gb, sg)

    def w_copy(jb, ob, sw):
        return pltpu.make_async_copy(ob, msg_hbm.at[pl.ds(ebase + jb * GB, GB)],
                                     sw)

    g_copy(0, gb0, sg0).start()
    g_copy(1, gb1, sg1).start()
    bufs = ((gb0, ob0, sg0, sw0), (gb1, ob1, sg1, sw1))

    @pl.loop(0, NB_B, step=2)
    def _pair(j):
        for b in range(2):
            jb = j + b
            gb, ob, sg, sw = bufs[b]
            g_copy(jb, gb, sg).wait()

            @pl.when(jb >= 2)
            def _():
                w_copy(jb - 2, ob, sw).wait()

            @pl.loop(0, GB)
            def _edge(jj):
                for kk in range(D // 32):
                    sl = pl.ds(kk * 16, 16)
                    u = plsc.bitcast(gb[jj, sl], jnp.bfloat16)
                    v = plsc.bitcast(gb[GB + jj, sl], jnp.bfloat16)
                    r = jnp.maximum(u + v, jnp.bfloat16(0.0))
                    ob[jj, sl] = plsc.bitcast(r, jnp.int32)

            @pl.when(jb + 2 < NB_B)
            def _():
                g_copy(jb + 2, gb, sg).start()

            w_copy(jb, ob, sw).start()

    w_copy(NB_B - 2, ob0, sw0).wait()
    w_copy(NB_B - 1, ob1, sw1).wait()


def _scatter_body(row_hbm, t_hbm, out_hbm, rb0, rb1, eid_buf, acc,
                  gr0, gr1, st0, st1, sr0, sr1, sg0, sg1, sw):
    wid = lax.axis_index("s") * NC + lax.axis_index("c")
    lo = wid * CHUNK

    @pl.loop(0, CHUNK + 1)
    def _init(r):
        for k in range(D // 16):
            acc[r, pl.ds(k * 16, 16)] = jnp.full((16,), NEG, jnp.float32)

    def r_copy(cb, rb, sr):
        return pltpu.make_async_copy(row_hbm.at[pl.ds(cb * CD, CD)], rb, sr)

    def fire(j, grb, stb, sgb):
        @pl.loop(0, GD // 16)
        def _s(gg):
            pk = eid_buf[pl.ds(j * GD + gg * 16, 16)]
            stb[pl.ds(gg * 16, 16)] = pk >> 9

        pltpu.async_copy(t_hbm.at[stb], grb, sgb)

    def wait_g(grb, stb, sgb):
        pltpu.make_async_copy(t_hbm.at[stb], grb, sgb).wait()

    def proc(j, grb):
        @pl.loop(0, GD // 16)
        def _g(gg):
            pk = eid_buf[pl.ds(j * GD + gg * 16, 16)]
            nidv = pk & 511
            nids = [nidv[i] for i in range(16)]

            @pl.loop(0, D // 16, unroll=4)
            def _k(k):
                sl = pl.ds(k * 16, 16)
                for jj in range(16):
                    r = gg * 16 + jj
                    acc[nids[jj], sl] = jnp.maximum(acc[nids[jj], sl],
                                                    grb[r, sl])

    r_copy(0, rb0, sr0).start()
    r_copy(1, rb1, sr1).start()

    @pl.loop(0, NCH, step=2)
    def _cpair(c):
        for b in range(2):
            cb = c + b
            rb = (rb0, rb1)[b]
            srb = (sr0, sr1)[b]
            r_copy(cb, rb, srb).wait()
            zero_off = jnp.zeros((16,), jnp.int32)
            eids0 = lax.iota(jnp.int32, 16) + cb * CD

            with jax.named_scope("d_filter"):
                @pl.loop(0, CD // 16, init_carry=(zero_off, eids0), unroll=2)
                def _filter(g, carry):
                    off_v, eids = carry
                    rows = rb[pl.ds(g * 16, 16)]
                    m = (rows >= lo) & (rows < lo + CHUNK)
                    pos = plsc.cumsum(m.astype(jnp.int32))
                    idxv = off_v + pos - 1
                    packed = (eids << 9) | (rows - lo)
                    plsc.store_scatter(eid_buf, [idxv], packed, mask=m)
                    cnt = plsc.all_reduce_population_count(m)
                    return (off_v + cnt, eids + 16)

                off_v, _ = _filter
                m_cnt = off_v[0]

            @pl.when(cb + 2 < NCH)
            def _():
                r_copy(cb + 2, rb, srb).start()

            sent = jnp.full((16,), CHUNK, jnp.int32)
            for p_ in range(GD // 16):
                eid_buf[pl.ds(m_cnt + p_ * 16, 16)] = sent
            nb = (m_cnt + (GD - 1)) >> 6

            with jax.named_scope("d_fire01"):
                @pl.when(nb > 0)
                def _():
                    fire(0, gr0, st0, sg0)

                @pl.when(nb > 1)
                def _():
                    fire(1, gr1, st1, sg1)

            with jax.named_scope("d_batch"):
                @pl.loop(0, nb)
                def _batch(j):
                    even = (j & 1) == 0

                    @pl.when(even)
                    def _():
                        with jax.named_scope("d_wait"):
                            wait_g(gr0, st0, sg0)
                        with jax.named_scope("d_proc"):
                            proc(j, gr0)

                        @pl.when(j + 2 < nb)
                        def _():
                            fire(j + 2, gr0, st0, sg0)

                    @pl.when(jnp.logical_not(even))
                    def _():
                        with jax.named_scope("d_wait"):
                            wait_g(gr1, st1, sg1)
                        with jax.named_scope("d_proc"):
                            proc(j, gr1)

                        @pl.when(j + 2 < nb)
                        def _():
                            fire(j + 2, gr1, st1, sg1)

    @pl.loop(0, CHUNK)
    def _fin(r):
        for k in range(D // 16):
            sl = pl.ds(k * 16, 16)
            a = acc[r, sl]
            acc[r, sl] = jnp.where(a > NEG_THRESH, a, 0.0)

    pltpu.async_copy(acc.at[pl.ds(0, CHUNK)], out_hbm.at[pl.ds(lo, CHUNK)],
                     sw).wait()


def kernel(x, edge_index, W1, b1, W2, b2):
    row = edge_index[0].astype(jnp.int32)
    col = edge_index[1].astype(jnp.int32)
    b1r = b1.reshape(1, D).astype(jnp.float32)
    b2r = b2.reshape(1, D).astype(jnp.float32)

    uv = pl.pallas_call(
        _uv_body,
        grid=(GA,),
        in_specs=[
            pl.BlockSpec((BN, D), lambda i: (lax.rem(i, GA // 2), 0)),
            pl.BlockSpec((2 * D, D), lambda i: (0, 0)),
            pl.BlockSpec((1, D), lambda i: (0, 0)),
        ],
        out_specs=pl.BlockSpec((BN, D), lambda i: (i, 0)),
        out_shape=jax.ShapeDtypeStruct((2 * N, D), jnp.bfloat16),
    )(x, W1, b1r)

    pad = jnp.zeros((EPAD - E,), jnp.int32)
    row_p = jnp.concatenate([row, pad]).reshape(-1, GB)
    col_p = jnp.concatenate([col, pad]).reshape(-1, GB) + N
    ids2 = jnp.concatenate([row_p, col_p], axis=1).reshape(-1)

    mesh = plsc.VectorSubcoreMesh(
        core_axis_name="c", subcore_axis_name="s", num_cores=NC, num_subcores=NS
    )

    uv_i32 = jax.lax.bitcast_convert_type(
        uv.reshape(2 * N, D // 2, 2), jnp.int32)

    msg = pl.kernel(
        _gather_body,
        out_type=jax.ShapeDtypeStruct((EPAD, D // 2), jnp.int32),
        mesh=mesh,
        compiler_params=pltpu.CompilerParams(needs_layout_passes=False),
        scratch_types=[
            pltpu.VMEM((2 * EPT,), jnp.int32),
            pltpu.VMEM((2 * GB, D // 2), jnp.int32),
            pltpu.VMEM((2 * GB, D // 2), jnp.int32),
            pltpu.VMEM((GB, D // 2), jnp.int32),
            pltpu.VMEM((GB, D // 2), jnp.int32),
            pltpu.SemaphoreType.DMA,
            pltpu.SemaphoreType.DMA,
            pltpu.SemaphoreType.DMA,
            pltpu.SemaphoreType.DMA,
        ],
    )(ids2, uv_i32)

    msg_bf = jax.lax.bitcast_convert_type(msg, jnp.bfloat16).reshape(EPAD, D)
    t = pl.pallas_call(
        _mlp2_body,
        grid=(EPAD // BE,),
        in_specs=[
            pl.BlockSpec((BE, D), lambda i: (i, 0)),
            pl.BlockSpec((D, D), lambda i: (0, 0)),
            pl.BlockSpec((1, D), lambda i: (0, 0)),
        ],
        out_specs=pl.BlockSpec((BE, D), lambda i: (i, 0)),
        out_shape=jax.ShapeDtypeStruct((EPAD, D), jnp.float32),
    )(msg_bf, W2, b2r)

    out_pad = pl.kernel(
        _scatter_body,
        out_type=jax.ShapeDtypeStruct((NPAD, D), jnp.float32),
        mesh=mesh,
        compiler_params=pltpu.CompilerParams(needs_layout_passes=False),
        scratch_types=[
            pltpu.VMEM((CD,), jnp.int32),
            pltpu.VMEM((CD,), jnp.int32),
            pltpu.VMEM((CD + GD,), jnp.int32),
            pltpu.VMEM((CHUNK + 1, D), jnp.float32),
            pltpu.VMEM((GD, D), jnp.float32),
            pltpu.VMEM((GD, D), jnp.float32),
            pltpu.VMEM((GD,), jnp.int32),
            pltpu.VMEM((GD,), jnp.int32),
            pltpu.SemaphoreType.DMA,
            pltpu.SemaphoreType.DMA,
            pltpu.SemaphoreType.DMA,
            pltpu.SemaphoreType.DMA,
            pltpu.SemaphoreType.DMA,
        ],
    )(row, t)

    return out_pad[:N]


# speedup vs baseline: 2.5653x; 2.5556x over previous
import numpy as np

import jax
import jax.numpy as jnp
from jax import lax
from jax.experimental import pallas as pl
from jax.experimental.pallas import tpu as pltpu
from jax.experimental.pallas import tpu_sc as plsc

N = 10000
E = 160000
D = 256
HW = D // 2

NC, NS = 2, 16
NW = NC * NS

BN = 400
GA = 2 * (N // BN)

EPT = 5120
EPAD = NW * EPT
GB = 64
NB_B = EPT // GB

BE = 2048

CHUNK = 320
NPAD = NW * CHUNK
CD = 4000
SUPER = 20000
NSUP = E // SUPER
NCH = E // CD
GD = 64

NEG = -3.0e38
_NEG_H = int(np.float32(NEG).view(np.int32)) >> 16 & 0xFFFF
NEG_WORD = np.int32(np.uint32((_NEG_H << 16) | _NEG_H).view(np.int32))
NEG_THRESH = -1.0e37


def _uv_body(x_ref, w1_ref, b1_ref, uv_ref):
    p = pl.program_id(0) // (GA // 2)
    w1a = w1_ref[:D, :]
    w1b = w1_ref[D:, :]
    w = jnp.where(p == 0, w1a - w1b, w1b)
    b = jnp.where(p == 0, b1_ref[...], jnp.zeros_like(b1_ref[...]))
    r = jnp.dot(x_ref[...], w, preferred_element_type=jnp.float32) + b
    uv_ref[...] = r.astype(jnp.bfloat16)


def _rne16(u):
    return u + jnp.int32(0x7FFF) + ((u >> 16) & 1)


def _mlp2_body(m_ref, we_ref, wo_ref, be_ref, bo_ref, t_ref):
    mi = m_ref[...]
    me = pltpu.bitcast(mi << 16, jnp.float32)
    mo = pltpu.bitcast(mi & jnp.int32(-65536), jnp.float32)
    mcat = jnp.concatenate([me, mo], axis=1)
    te = jnp.dot(mcat, we_ref[...], preferred_element_type=jnp.float32)
    to = jnp.dot(mcat, wo_ref[...], preferred_element_type=jnp.float32)
    te = te + be_ref[...]
    to = to + bo_ref[...]
    ue = pltpu.bitcast(te, jnp.int32)
    uo = pltpu.bitcast(to, jnp.int32)
    lo = (_rne16(ue) >> 16) & jnp.int32(0xFFFF)
    hi = _rne16(uo) & jnp.int32(-65536)
    t_ref[...] = hi | lo


def _gather_body(ids_hbm, uv_hbm, msg_hbm, ids_v, gb0, gb1, ob0, ob1,
                 sg0, sg1, sw0, sw1):
    wid = lax.axis_index("s") * NC + lax.axis_index("c")
    ibase = wid * (2 * EPT)
    ebase = wid * EPT
    pltpu.sync_copy(ids_hbm.at[pl.ds(ibase, 2 * EPT)], ids_v)

    def g_copy(jb, gb, sg):
        idx = ids_v.at[pl.ds(jb * (2 * GB), 2 * GB)]
        return pltpu.make_async_copy(uv_hbm.at[idx], gb, sg)

    def w_copy(jb, ob, sw):
        return pltpu.make_async_copy(ob, msg_hbm.at[pl.ds(ebase + jb * GB, GB)],
                                     sw)

    g_copy(0, gb0, sg0).start()
    g_copy(1, gb1, sg1).start()
    bufs = ((gb0, ob0, sg0, sw0), (gb1, ob1, sg1, sw1))

    @pl.loop(0, NB_B, step=2)
    def _pair(j):
        for b in range(2):
            jb = j + b
            gb, ob, sg, sw = bufs[b]
            g_copy(jb, gb, sg).wait()

            @pl.when(jb >= 2)
            def _():
                w_copy(jb - 2, ob, sw).wait()

            @pl.loop(0, GB)
            def _edge(jj):
                for kk in range(HW // 16):
                    sl = pl.ds(kk * 16, 16)
                    u = plsc.bitcast(gb[jj, sl], jnp.bfloat16)
                    v = plsc.bitcast(gb[GB + jj, sl], jnp.bfloat16)
                    r = jnp.maximum(u + v, jnp.bfloat16(0.0))
                    ob[jj, sl] = plsc.bitcast(r, jnp.int32)

            @pl.when(jb + 2 < NB_B)
            def _():
                g_copy(jb + 2, gb, sg).start()

            w_copy(jb, ob, sw).start()

    w_copy(NB_B - 2, ob0, sw0).wait()
    w_copy(NB_B - 1, ob1, sw1).wait()


def _scatter_body(row_hbm, t_hbm, out_hbm, rb0, rb1, eb0, eb1, acc,
                  gr0, gr1, gr2, gr3, st0, st1, st2, st3,
                  sr0, sr1, sg0, sg1, sg2, sg3, sw):
    wid = lax.axis_index("s") * NC + lax.axis_index("c")
    lo = wid * CHUNK
    rbs = (rb0, rb1)
    srs = (sr0, sr1)
    ebs = (eb0, eb1)
    grs = ((gr0, st0, sg0), (gr1, st1, sg1), (gr2, st2, sg2), (gr3, st3, sg3))

    @pl.loop(0, CHUNK + 1)
    def _init(r):
        for k in range(HW // 16):
            acc[r, pl.ds(k * 16, 16)] = jnp.full((16,), NEG_WORD, jnp.int32)

    def r_copy(ci, rb, sr):
        return pltpu.make_async_copy(row_hbm.at[pl.ds(ci * CD, CD)], rb, sr)

    def filter_super(s, sb, eb):
        off_v = jnp.zeros((16,), jnp.int32)
        for sub in range(SUPER // CD):
            ci = s * (SUPER // CD) + sub
            rb = rbs[(sb * (SUPER // CD) + sub) % 2]
            sr = srs[(sb * (SUPER // CD) + sub) % 2]
            r_copy(ci, rb, sr).wait()
            eids0 = lax.iota(jnp.int32, 16) + ci * CD

            @pl.loop(0, CD // 16, init_carry=(off_v, eids0), unroll=2)
            def _filter(g, carry):
                o_v, eids = carry
                rows = rb[pl.ds(g * 16, 16)]
                m = (rows >= lo) & (rows < lo + CHUNK)
                pos = plsc.cumsum(m.astype(jnp.int32))
                idxv = o_v + pos - 1
                packed = (eids << 9) | (rows - lo)
                plsc.store_scatter(eb, [idxv], packed, mask=m)
                cnt = plsc.all_reduce_population_count(m)
                return (o_v + cnt, eids + 16)

            off_v, _ = _filter

            @pl.when(ci + 2 < NCH)
            def _():
                r_copy(ci + 2, rb, sr).start()

        m_cnt = off_v[0]
        sent = jnp.full((16,), CHUNK, jnp.int32)
        for p_ in range(GD // 16):
            eb[pl.ds(m_cnt + p_ * 16, 16)] = sent
        return m_cnt

    def fire(eb, j, grb, stb, sgb):
        @pl.loop(0, GD // 16)
        def _s(gg):
            pk = eb[pl.ds(j * GD + gg * 16, 16)]
            stb[pl.ds(gg * 16, 16)] = pk >> 9

        pltpu.async_copy(t_hbm.at[stb], grb, sgb)

    def proc(eb, j, grb):
        @pl.loop(0, GD // 16)
        def _g(gg):
            pk = eb[pl.ds(j * GD + gg * 16, 16)]
            nidv = pk & 511
            nids = [nidv[i] for i in range(16)]

            @pl.loop(0, HW // 16)
            def _k(k):
                sl = pl.ds(k * 16, 16)
                for jj in range(16):
                    r = gg * 16 + jj
                    a = plsc.bitcast(acc[nids[jj], sl], jnp.bfloat16)
                    g = plsc.bitcast(grb[r, sl], jnp.bfloat16)
                    acc[nids[jj], sl] = plsc.bitcast(jnp.maximum(a, g),
                                                     jnp.int32)

    r_copy(0, rb0, sr0).start()
    r_copy(1, rb1, sr1).start()

    @pl.loop(0, NSUP, step=2)
    def _super(s0):
        for sb in range(2):
            s = s0 + sb
            eb = ebs[sb]
            m_cnt = filter_super(s, sb, eb)
            nb = (m_cnt + (GD - 1)) >> 6
            for jf in range(4):
                grb, stb, sgb = grs[jf]

                @pl.when(nb > jf)
                def _():
                    fire(eb, jf, grb, stb, sgb)

            @pl.loop(0, nb)
            def _batch(j):
                par = j & 3
                for jp in range(4):
                    grb, stb, sgb = grs[jp]

                    @pl.when(par == jp)
                    def _():
                        pltpu.make_async_copy(t_hbm.at[stb], grb, sgb).wait()
                        proc(eb, j, grb)

                        @pl.when(j + 4 < nb)
                        def _():
                            fire(eb, j + 4, grb, stb, sgb)

    @pl.loop(0, CHUNK)
    def _fin(r):
        for k in range(HW // 16):
            sl = pl.ds(k * 16, 16)
            a = plsc.bitcast(acc[r, sl], jnp.bfloat16)
            m = a > jnp.bfloat16(NEG_THRESH)
            sel = jnp.where(m, a, jnp.bfloat16(0.0))
            acc[r, sl] = plsc.bitcast(sel, jnp.int32)

    pltpu.async_copy(acc.at[pl.ds(0, CHUNK)], out_hbm.at[pl.ds(lo, CHUNK)],
                     sw).wait()


def kernel(x, edge_index, W1, b1, W2, b2):
    row = edge_index[0].astype(jnp.int32)
    col = edge_index[1].astype(jnp.int32)
    b1r = b1.reshape(1, D).astype(jnp.float32)

    uv = pl.pallas_call(
        _uv_body,
        grid=(GA,),
        in_specs=[
            pl.BlockSpec((BN, D), lambda i: (lax.rem(i, GA // 2), 0)),
            pl.BlockSpec((2 * D, D), lambda i: (0, 0)),
            pl.BlockSpec((1, D), lambda i: (0, 0)),
        ],
        out_specs=pl.BlockSpec((BN, D), lambda i: (i, 0)),
        out_shape=jax.ShapeDtypeStruct((2 * N, D), jnp.bfloat16),
    )(x, W1, b1r)

    pad = jnp.zeros((EPAD - E,), jnp.int32)
    row_p = jnp.concatenate([row, pad]).reshape(-1, GB)
    col_p = jnp.concatenate([col, pad]).reshape(-1, GB) + N
    ids2 = jnp.concatenate([row_p, col_p], axis=1).reshape(-1)

    mesh = plsc.VectorSubcoreMesh(
        core_axis_name="c", subcore_axis_name="s", num_cores=NC, num_subcores=NS
    )

    uv_i32 = jax.lax.bitcast_convert_type(
        uv.reshape(2 * N, HW, 2), jnp.int32)

    msg = pl.kernel(
        _gather_body,
        out_type=jax.ShapeDtypeStruct((EPAD, HW), jnp.int32),
        mesh=mesh,
        compiler_params=pltpu.CompilerParams(needs_layout_passes=False),
        scratch_types=[
            pltpu.VMEM((2 * EPT,), jnp.int32),
            pltpu.VMEM((2 * GB, HW), jnp.int32),
            pltpu.VMEM((2 * GB, HW), jnp.int32),
            pltpu.VMEM((GB, HW), jnp.int32),
            pltpu.VMEM((GB, HW), jnp.int32),
            pltpu.SemaphoreType.DMA,
            pltpu.SemaphoreType.DMA,
            pltpu.SemaphoreType.DMA,
            pltpu.SemaphoreType.DMA,
        ],
    )(ids2, uv_i32)

    we = jnp.concatenate([W2[0::2, 0::2], W2[1::2, 0::2]], axis=0)
    wo = jnp.concatenate([W2[0::2, 1::2], W2[1::2, 1::2]], axis=0)
    be = b2[0::2].reshape(1, HW).astype(jnp.float32)
    bo = b2[1::2].reshape(1, HW).astype(jnp.float32)
    t = pl.pallas_call(
        _mlp2_body,
        grid=(EPAD // BE,),
        in_specs=[
            pl.BlockSpec((BE, HW), lambda i: (i, 0)),
            pl.BlockSpec((D, HW), lambda i: (0, 0)),
            pl.BlockSpec((D, HW), lambda i: (0, 0)),
            pl.BlockSpec((1, HW), lambda i: (0, 0)),
            pl.BlockSpec((1, HW), lambda i: (0, 0)),
        ],
        out_specs=pl.BlockSpec((BE, HW), lambda i: (i, 0)),
        out_shape=jax.ShapeDtypeStruct((EPAD, HW), jnp.int32),
    )(msg, we, wo, be, bo)

    out_pad = pl.kernel(
        _scatter_body,
        out_type=jax.ShapeDtypeStruct((NPAD, HW), jnp.int32),
        mesh=mesh,
        compiler_params=pltpu.CompilerParams(needs_layout_passes=False),
        scratch_types=[
            pltpu.VMEM((CD,), jnp.int32),
            pltpu.VMEM((CD,), jnp.int32),
            pltpu.VMEM((SUPER + GD,), jnp.int32),
            pltpu.VMEM((SUPER + GD,), jnp.int32),
            pltpu.VMEM((CHUNK + 1, HW), jnp.int32),
            pltpu.VMEM((GD, HW), jnp.int32),
            pltpu.VMEM((GD, HW), jnp.int32),
            pltpu.VMEM((GD, HW), jnp.int32),
            pltpu.VMEM((GD, HW), jnp.int32),
            pltpu.VMEM((GD,), jnp.int32),
            pltpu.VMEM((GD,), jnp.int32),
            pltpu.VMEM((GD,), jnp.int32),
            pltpu.VMEM((GD,), jnp.int32),
            pltpu.SemaphoreType.DMA,
            pltpu.SemaphoreType.DMA,
            pltpu.SemaphoreType.DMA,
            pltpu.SemaphoreType.DMA,
            pltpu.SemaphoreType.DMA,
            pltpu.SemaphoreType.DMA,
            pltpu.SemaphoreType.DMA,
        ],
    )(row, t)

    out_bf = jax.lax.bitcast_convert_type(out_pad, jnp.bfloat16)
    return out_bf.reshape(NPAD, D).astype(jnp.float32)[:N]


# speedup vs baseline: 2.6541x; 1.0346x over previous
import numpy as np

import jax
import jax.numpy as jnp
from jax import lax
from jax.experimental import pallas as pl
from jax.experimental.pallas import tpu as pltpu
from jax.experimental.pallas import tpu_sc as plsc

N = 10000
E = 160000
D = 256
HW = D // 2

NC, NS = 2, 16
NW = NC * NS

BN = 400
GA = 2 * (N // BN)

EPT = 5120
EPAD = NW * EPT
GB = 64
NB_B = EPT // GB

BE = 2048

CHUNK = 320
NPAD = NW * CHUNK
CD = 4000
SUPER = 40000
NSUP = E // SUPER
NCH = E // CD
GD = 64

NEG = -3.0e38
_NEG_H = int(np.float32(NEG).view(np.int32)) >> 16 & 0xFFFF
NEG_WORD = np.int32(np.uint32((_NEG_H << 16) | _NEG_H).view(np.int32))
NEG_THRESH = -1.0e37


def _uv_body(x_ref, w1_ref, b1_ref, uv_ref):
    p = pl.program_id(0) // (GA // 2)
    w1a = w1_ref[:D, :]
    w1b = w1_ref[D:, :]
    w = jnp.where(p == 0, w1a - w1b, w1b)
    b = jnp.where(p == 0, b1_ref[...], jnp.zeros_like(b1_ref[...]))
    r = jnp.dot(x_ref[...], w, preferred_element_type=jnp.float32) + b
    uv_ref[...] = r.astype(jnp.bfloat16)


def _rne16(u):
    return u + jnp.int32(0x7FFF) + ((u >> 16) & 1)


def _mlp2_body(m_ref, we_ref, wo_ref, be_ref, bo_ref, t_ref):
    mi = m_ref[...]
    me = pltpu.bitcast(mi << 16, jnp.float32)
    mo = pltpu.bitcast(mi & jnp.int32(-65536), jnp.float32)
    mcat = jnp.concatenate([me, mo], axis=1)
    te = jnp.dot(mcat, we_ref[...], preferred_element_type=jnp.float32)
    to = jnp.dot(mcat, wo_ref[...], preferred_element_type=jnp.float32)
    te = te + be_ref[...]
    to = to + bo_ref[...]
    ue = pltpu.bitcast(te, jnp.int32)
    uo = pltpu.bitcast(to, jnp.int32)
    lo = (_rne16(ue) >> 16) & jnp.int32(0xFFFF)
    hi = _rne16(uo) & jnp.int32(-65536)
    t_ref[...] = hi | lo


def _gather_body(ids_hbm, uv_hbm, msg_hbm, ids_v, gb0, gb1, gb2, gb3,
                 ob0, ob1, sg0, sg1, sg2, sg3, sw0, sw1):
    wid = lax.axis_index("s") * NC + lax.axis_index("c")
    ibase = wid * (2 * EPT)
    ebase = wid * EPT
    pltpu.sync_copy(ids_hbm.at[pl.ds(ibase, 2 * EPT)], ids_v)

    def g_copy(jb, gb, sg):
        idx = ids_v.at[pl.ds(jb * (2 * GB), 2 * GB)]
        return pltpu.make_async_copy(uv_hbm.at[idx], gb, sg)

    def w_copy(jb, ob, sw):
        return pltpu.make_async_copy(ob, msg_hbm.at[pl.ds(ebase + jb * GB, GB)],
                                     sw)

    gbs = ((gb0, sg0), (gb1, sg1), (gb2, sg2), (gb3, sg3))
    obs = ((ob0, sw0), (ob1, sw1))
    for jb in range(4):
        g_copy(jb, gbs[jb][0], gbs[jb][1]).start()

    @pl.loop(0, NB_B, step=4)
    def _quad(j):
        for b in range(4):
            jb = j + b
            gb, sg = gbs[b]
            ob, sw = obs[b % 2]
            g_copy(jb, gb, sg).wait()

            @pl.when(jb >= 2)
            def _():
                w_copy(jb - 2, ob, sw).wait()

            @pl.loop(0, GB)
            def _edge(jj):
                for kk in range(HW // 16):
                    sl = pl.ds(kk * 16, 16)
                    u = plsc.bitcast(gb[jj, sl], jnp.bfloat16)
                    v = plsc.bitcast(gb[GB + jj, sl], jnp.bfloat16)
                    r = jnp.maximum(u + v, jnp.bfloat16(0.0))
                    ob[jj, sl] = plsc.bitcast(r, jnp.int32)

            @pl.when(jb + 4 < NB_B)
            def _():
                g_copy(jb + 4, gb, sg).start()

            w_copy(jb, ob, sw).start()

    w_copy(NB_B - 2, ob0, sw0).wait()
    w_copy(NB_B - 1, ob1, sw1).wait()


def _scatter_body(row_hbm, t_hbm, out_hbm, rb0, rb1, eb0, acc,
                  gr0, gr1, gr2, gr3, st0, st1, st2, st3,
                  sr0, sr1, sg0, sg1, sg2, sg3, sw):
    wid = lax.axis_index("s") * NC + lax.axis_index("c")
    lo = wid * CHUNK
    rbs = (rb0, rb1)
    srs = (sr0, sr1)
    ebs = (eb0, eb0)
    grs = ((gr0, st0, sg0), (gr1, st1, sg1), (gr2, st2, sg2), (gr3, st3, sg3))

    @pl.loop(0, CHUNK + 1)
    def _init(r):
        for k in range(HW // 16):
            acc[r, pl.ds(k * 16, 16)] = jnp.full((16,), NEG_WORD, jnp.int32)

    def r_copy(ci, rb, sr):
        return pltpu.make_async_copy(row_hbm.at[pl.ds(ci * CD, CD)], rb, sr)

    def filter_super(s, sb, eb):
        off_v = jnp.zeros((16,), jnp.int32)
        for sub in range(SUPER // CD):
            ci = s * (SUPER // CD) + sub
            rb = rbs[(sb * (SUPER // CD) + sub) % 2]
            sr = srs[(sb * (SUPER // CD) + sub) % 2]
            r_copy(ci, rb, sr).wait()
            eids0 = lax.iota(jnp.int32, 16) + ci * CD

            @pl.loop(0, CD // 16, init_carry=(off_v, eids0), unroll=2)
            def _filter(g, carry):
                o_v, eids = carry
                rows = rb[pl.ds(g * 16, 16)]
                m = (rows >= lo) & (rows < lo + CHUNK)
                pos = plsc.cumsum(m.astype(jnp.int32))
                idxv = o_v + pos - 1
                packed = (eids << 9) | (rows - lo)
                plsc.store_scatter(eb, [idxv], packed, mask=m)
                cnt = plsc.all_reduce_population_count(m)
                return (o_v + cnt, eids + 16)

            off_v, _ = _filter

            @pl.when(ci + 2 < NCH)
            def _():
                r_copy(ci + 2, rb, sr).start()

        m_cnt = off_v[0]
        sent = jnp.full((16,), CHUNK, jnp.int32)
        for p_ in range(GD // 16):
            eb[pl.ds(m_cnt + p_ * 16, 16)] = sent
        return m_cnt

    def fire(eb, j, grb, stb, sgb):
        @pl.loop(0, GD // 16)
        def _s(gg):
            pk = eb[pl.ds(j * GD + gg * 16, 16)]
            stb[pl.ds(gg * 16, 16)] = pk >> 9

        pltpu.async_copy(t_hbm.at[stb], grb, sgb)

    def proc(eb, j, grb):
        @pl.loop(0, GD // 16)
        def _g(gg):
            pk = eb[pl.ds(j * GD + gg * 16, 16)]
            nidv = pk & 511
            nids = [nidv[i] for i in range(16)]

            @pl.loop(0, HW // 16)
            def _k(k):
                sl = pl.ds(k * 16, 16)
                for jj in range(16):
                    r = gg * 16 + jj
                    a = plsc.bitcast(acc[nids[jj], sl], jnp.bfloat16)
                    g = plsc.bitcast(grb[r, sl], jnp.bfloat16)
                    acc[nids[jj], sl] = plsc.bitcast(jnp.maximum(a, g),
                                                     jnp.int32)

    r_copy(0, rb0, sr0).start()
    r_copy(1, rb1, sr1).start()

    @pl.loop(0, NSUP, step=2)
    def _super(s0):
        for sb in range(2):
            s = s0 + sb
            eb = ebs[sb]
            m_cnt = filter_super(s, sb, eb)
            nb = (m_cnt + (GD - 1)) >> 6
            for jf in range(4):
                grb, stb, sgb = grs[jf]

                @pl.when(nb > jf)
                def _():
                    fire(eb, jf, grb, stb, sgb)

            @pl.loop(0, nb)
            def _batch(j):
                par = j & 3
                for jp in range(4):
                    grb, stb, sgb = grs[jp]

                    @pl.when(par == jp)
                    def _():
                        pltpu.make_async_copy(t_hbm.at[stb], grb, sgb).wait()
                        proc(eb, j, grb)

                        @pl.when(j + 4 < nb)
                        def _():
                            fire(eb, j + 4, grb, stb, sgb)

    @pl.loop(0, CHUNK)
    def _fin(r):
        for k in range(HW // 16):
            sl = pl.ds(k * 16, 16)
            a = plsc.bitcast(acc[r, sl], jnp.bfloat16)
            m = a > jnp.bfloat16(NEG_THRESH)
            sel = jnp.where(m, a, jnp.bfloat16(0.0))
            acc[r, sl] = plsc.bitcast(sel, jnp.int32)

    pltpu.async_copy(acc.at[pl.ds(0, CHUNK)], out_hbm.at[pl.ds(lo, CHUNK)],
                     sw).wait()


def kernel(x, edge_index, W1, b1, W2, b2):
    row = edge_index[0].astype(jnp.int32)
    col = edge_index[1].astype(jnp.int32)
    b1r = b1.reshape(1, D).astype(jnp.float32)

    uv = pl.pallas_call(
        _uv_body,
        grid=(GA,),
        in_specs=[
            pl.BlockSpec((BN, D), lambda i: (lax.rem(i, GA // 2), 0)),
            pl.BlockSpec((2 * D, D), lambda i: (0, 0)),
            pl.BlockSpec((1, D), lambda i: (0, 0)),
        ],
        out_specs=pl.BlockSpec((BN, D), lambda i: (i, 0)),
        out_shape=jax.ShapeDtypeStruct((2 * N, D), jnp.bfloat16),
    )(x, W1, b1r)

    pad = jnp.zeros((EPAD - E,), jnp.int32)
    row_p = jnp.concatenate([row, pad]).reshape(-1, GB)
    col_p = jnp.concatenate([col, pad]).reshape(-1, GB) + N
    ids2 = jnp.concatenate([row_p, col_p], axis=1).reshape(-1)

    mesh = plsc.VectorSubcoreMesh(
        core_axis_name="c", subcore_axis_name="s", num_cores=NC, num_subcores=NS
    )

    uv_i32 = jax.lax.bitcast_convert_type(
        uv.reshape(2 * N, HW, 2), jnp.int32)

    msg = pl.kernel(
        _gather_body,
        out_type=jax.ShapeDtypeStruct((EPAD, HW), jnp.int32),
        mesh=mesh,
        compiler_params=pltpu.CompilerParams(needs_layout_passes=False),
        scratch_types=[
            pltpu.VMEM((2 * EPT,), jnp.int32),
            pltpu.VMEM((2 * GB, HW), jnp.int32),
            pltpu.VMEM((2 * GB, HW), jnp.int32),
            pltpu.VMEM((2 * GB, HW), jnp.int32),
            pltpu.VMEM((2 * GB, HW), jnp.int32),
            pltpu.VMEM((GB, HW), jnp.int32),
            pltpu.VMEM((GB, HW), jnp.int32),
            pltpu.SemaphoreType.DMA,
            pltpu.SemaphoreType.DMA,
            pltpu.SemaphoreType.DMA,
            pltpu.SemaphoreType.DMA,
            pltpu.SemaphoreType.DMA,
            pltpu.SemaphoreType.DMA,
        ],
    )(ids2, uv_i32)

    we = jnp.concatenate([W2[0::2, 0::2], W2[1::2, 0::2]], axis=0)
    wo = jnp.concatenate([W2[0::2, 1::2], W2[1::2, 1::2]], axis=0)
    be = b2[0::2].reshape(1, HW).astype(jnp.float32)
    bo = b2[1::2].reshape(1, HW).astype(jnp.float32)
    t = pl.pallas_call(
        _mlp2_body,
        grid=(EPAD // BE,),
        in_specs=[
            pl.BlockSpec((BE, HW), lambda i: (i, 0)),
            pl.BlockSpec((D, HW), lambda i: (0, 0)),
            pl.BlockSpec((D, HW), lambda i: (0, 0)),
            pl.BlockSpec((1, HW), lambda i: (0, 0)),
            pl.BlockSpec((1, HW), lambda i: (0, 0)),
        ],
        out_specs=pl.BlockSpec((BE, HW), lambda i: (i, 0)),
        out_shape=jax.ShapeDtypeStruct((EPAD, HW), jnp.int32),
    )(msg, we, wo, be, bo)

    out_pad = pl.kernel(
        _scatter_body,
        out_type=jax.ShapeDtypeStruct((NPAD, HW), jnp.int32),
        mesh=mesh,
        compiler_params=pltpu.CompilerParams(needs_layout_passes=False),
        scratch_types=[
            pltpu.VMEM((CD,), jnp.int32),
            pltpu.VMEM((CD,), jnp.int32),
            pltpu.VMEM((SUPER + GD,), jnp.int32),
            pltpu.VMEM((CHUNK + 1, HW), jnp.int32),
            pltpu.VMEM((GD, HW), jnp.int32),
            pltpu.VMEM((GD, HW), jnp.int32),
            pltpu.VMEM((GD, HW), jnp.int32),
            pltpu.VMEM((GD, HW), jnp.int32),
            pltpu.VMEM((GD,), jnp.int32),
            pltpu.VMEM((GD,), jnp.int32),
            pltpu.VMEM((GD,), jnp.int32),
            pltpu.VMEM((GD,), jnp.int32),
            pltpu.SemaphoreType.DMA,
            pltpu.SemaphoreType.DMA,
            pltpu.SemaphoreType.DMA,
            pltpu.SemaphoreType.DMA,
            pltpu.SemaphoreType.DMA,
            pltpu.SemaphoreType.DMA,
            pltpu.SemaphoreType.DMA,
        ],
    )(row, t)

    out_bf = jax.lax.bitcast_convert_type(out_pad, jnp.bfloat16)
    return out_bf.reshape(NPAD, D).astype(jnp.float32)[:N]


# speedup vs baseline: 3.1065x; 1.1705x over previous
import numpy as np

import jax
import jax.numpy as jnp
from jax import lax
from jax.experimental import pallas as pl
from jax.experimental.pallas import tpu as pltpu
from jax.experimental.pallas import tpu_sc as plsc

N = 10000
E = 160000
D = 256
HW = D // 2

NC, NS = 2, 16
NW = NC * NS

BN = 400
GA = 2 * (N // BN)

EPT = 5120
EPAD = NW * EPT
GB = 64
NB_B = EPT // GB

BE = 2048

CHUNK = 320
NPAD = NW * CHUNK
CD = 4000
SUPER = 40000
NSUP = E // SUPER
NCH = E // CD
GD = 64

NEG = -3.0e38
_NEG_H = int(np.float32(NEG).view(np.int32)) >> 16 & 0xFFFF
NEG_WORD = np.int32(np.uint32((_NEG_H << 16) | _NEG_H).view(np.int32))
NEG_THRESH = -1.0e37


def _uv_body(x_ref, w1e_ref, w1o_ref, b1e_ref, b1o_ref, uv_ref):
    p = pl.program_id(0) // (GA // 2)
    we = jnp.where(p == 0, w1e_ref[:D, :] - w1e_ref[D:, :], w1e_ref[D:, :])
    wo = jnp.where(p == 0, w1o_ref[:D, :] - w1o_ref[D:, :], w1o_ref[D:, :])
    bb_e = jnp.where(p == 0, b1e_ref[...], jnp.zeros_like(b1e_ref[...]))
    bb_o = jnp.where(p == 0, b1o_ref[...], jnp.zeros_like(b1o_ref[...]))
    re = jnp.dot(x_ref[...], we, preferred_element_type=jnp.float32) + bb_e
    ro = jnp.dot(x_ref[...], wo, preferred_element_type=jnp.float32) + bb_o
    ue = pltpu.bitcast(re, jnp.int32)
    uo = pltpu.bitcast(ro, jnp.int32)
    lo = (_rne16(ue) >> 16) & jnp.int32(0xFFFF)
    hi = _rne16(uo) & jnp.int32(-65536)
    uv_ref[...] = hi | lo


def _rne16(u):
    return u + jnp.int32(0x7FFF) + ((u >> 16) & 1)


def _mlp2_body(m_ref, we_ref, wo_ref, be_ref, bo_ref, t_ref):
    mi = m_ref[...]
    me = pltpu.bitcast(mi << 16, jnp.float32)
    mo = pltpu.bitcast(mi & jnp.int32(-65536), jnp.float32)
    mcat = jnp.concatenate([me, mo], axis=1).astype(jnp.bfloat16)
    te = jnp.dot(mcat, we_ref[...], preferred_element_type=jnp.float32)
    to = jnp.dot(mcat, wo_ref[...], preferred_element_type=jnp.float32)
    te = te + be_ref[...]
    to = to + bo_ref[...]
    ue = pltpu.bitcast(te, jnp.int32)
    uo = pltpu.bitcast(to, jnp.int32)
    lo = (_rne16(ue) >> 16) & jnp.int32(0xFFFF)
    hi = _rne16(uo) & jnp.int32(-65536)
    t_ref[...] = hi | lo


def _gather_body(row_hbm, col_hbm, uv_hbm, msg_hbm, idr, idc,
                 st0, st1, st2, st3, gb0, gb1, gb2, gb3,
                 ob0, ob1, sg0, sg1, sg2, sg3, sw0, sw1):
    wid = lax.axis_index("s") * NC + lax.axis_index("c")
    ebase = wid * EPT
    pltpu.sync_copy(row_hbm.at[pl.ds(ebase, EPT)], idr)
    pltpu.sync_copy(col_hbm.at[pl.ds(ebase, EPT)], idc)
    sts = (st0, st1, st2, st3)

    def stage_idx(jb, st):
        @pl.loop(0, GB // 16)
        def _s(g):
            sl = pl.ds(jb * GB + g * 16, 16)
            st[pl.ds(g * 16, 16)] = idr[sl]
            st[pl.ds(GB + g * 16, 16)] = idc[sl] + N

    def g_copy(jb, gb, sg, st):
        return pltpu.make_async_copy(uv_hbm.at[st], gb, sg)

    def w_copy(jb, ob, sw):
        return pltpu.make_async_copy(ob, msg_hbm.at[pl.ds(ebase + jb * GB, GB)],
                                     sw)

    gbs = ((gb0, sg0), (gb1, sg1), (gb2, sg2), (gb3, sg3))
    obs = ((ob0, sw0), (ob1, sw1))
    for jb in range(4):
        stage_idx(jb, sts[jb])
        g_copy(jb, gbs[jb][0], gbs[jb][1], sts[jb]).start()

    @pl.loop(0, NB_B, step=4)
    def _quad(j):
        for b in range(4):
            jb = j + b
            gb, sg = gbs[b]
            st = sts[b]
            ob, sw = obs[b % 2]
            g_copy(jb, gb, sg, st).wait()

            @pl.when(jb >= 2)
            def _():
                w_copy(jb - 2, ob, sw).wait()

            @pl.loop(0, GB)
            def _edge(jj):
                for kk in range(HW // 16):
                    sl = pl.ds(kk * 16, 16)
                    u = plsc.bitcast(gb[jj, sl], jnp.bfloat16)
                    v = plsc.bitcast(gb[GB + jj, sl], jnp.bfloat16)
                    r = jnp.maximum(u + v, jnp.bfloat16(0.0))
                    ob[jj, sl] = plsc.bitcast(r, jnp.int32)

            @pl.when(jb + 4 < NB_B)
            def _():
                stage_idx(jb + 4, st)
                g_copy(jb + 4, gb, sg, st).start()

            w_copy(jb, ob, sw).start()

    w_copy(NB_B - 2, ob0, sw0).wait()
    w_copy(NB_B - 1, ob1, sw1).wait()


def _scatter_body(row_hbm, t_hbm, out_hbm, rb0, rb1, eb0, acc,
                  gr0, gr1, gr2, gr3, st0, st1, st2, st3,
                  sr0, sr1, sg0, sg1, sg2, sg3, sw):
    wid = lax.axis_index("s") * NC + lax.axis_index("c")
    lo = wid * CHUNK
    rbs = (rb0, rb1)
    srs = (sr0, sr1)
    ebs = (eb0, eb0)
    grs = ((gr0, st0, sg0), (gr1, st1, sg1), (gr2, st2, sg2), (gr3, st3, sg3))

    @pl.loop(0, CHUNK + 1)
    def _init(r):
        for k in range(HW // 16):
            acc[r, pl.ds(k * 16, 16)] = jnp.full((16,), NEG_WORD, jnp.int32)

    def r_copy(ci, rb, sr):
        return pltpu.make_async_copy(row_hbm.at[pl.ds(ci * CD, CD)], rb, sr)

    def filter_super(s, sb, eb):
        off_v = jnp.zeros((16,), jnp.int32)
        for sub in range(SUPER // CD):
            ci = s * (SUPER // CD) + sub
            rb = rbs[(sb * (SUPER // CD) + sub) % 2]
            sr = srs[(sb * (SUPER // CD) + sub) % 2]
            r_copy(ci, rb, sr).wait()
            eids0 = lax.iota(jnp.int32, 16) + ci * CD

            @pl.loop(0, CD // 16, init_carry=(off_v, eids0), unroll=2)
            def _filter(g, carry):
                o_v, eids = carry
                rows = rb[pl.ds(g * 16, 16)]
                m = (rows >= lo) & (rows < lo + CHUNK)
                pos = plsc.cumsum(m.astype(jnp.int32))
                idxv = o_v + pos - 1
                packed = (eids << 9) | (rows - lo)
                plsc.store_scatter(eb, [idxv], packed, mask=m)
                cnt = plsc.all_reduce_population_count(m)
                return (o_v + cnt, eids + 16)

            off_v, _ = _filter

            @pl.when(ci + 2 < NCH)
            def _():
                r_copy(ci + 2, rb, sr).start()

        m_cnt = off_v[0]
        sent = jnp.full((16,), CHUNK, jnp.int32)
        for p_ in range(GD // 16):
            eb[pl.ds(m_cnt + p_ * 16, 16)] = sent
        return m_cnt

    def fire(eb, j, grb, stb, sgb):
        @pl.loop(0, GD // 16)
        def _s(gg):
            pk = eb[pl.ds(j * GD + gg * 16, 16)]
            stb[pl.ds(gg * 16, 16)] = pk >> 9

        pltpu.async_copy(t_hbm.at[stb], grb, sgb)

    def proc(eb, j, grb):
        @pl.loop(0, GD // 16)
        def _g(gg):
            pk = eb[pl.ds(j * GD + gg * 16, 16)]
            nidv = pk & 511
            nids = [nidv[i] for i in range(16)]

            @pl.loop(0, HW // 16)
            def _k(k):
                sl = pl.ds(k * 16, 16)
                for jj in range(16):
                    r = gg * 16 + jj
                    a = plsc.bitcast(acc[nids[jj], sl], jnp.bfloat16)
                    g = plsc.bitcast(grb[r, sl], jnp.bfloat16)
                    acc[nids[jj], sl] = plsc.bitcast(jnp.maximum(a, g),
                                                     jnp.int32)

    r_copy(0, rb0, sr0).start()
    r_copy(1, rb1, sr1).start()

    @pl.loop(0, NSUP, step=2)
    def _super(s0):
        for sb in range(2):
            s = s0 + sb
            eb = ebs[sb]
            m_cnt = filter_super(s, sb, eb)
            nb = (m_cnt + (GD - 1)) >> 6
            for jf in range(4):
                grb, stb, sgb = grs[jf]

                @pl.when(nb > jf)
                def _():
                    fire(eb, jf, grb, stb, sgb)

            @pl.loop(0, nb)
            def _batch(j):
                par = j & 3
                for jp in range(4):
                    grb, stb, sgb = grs[jp]

                    @pl.when(par == jp)
                    def _():
                        pltpu.make_async_copy(t_hbm.at[stb], grb, sgb).wait()
                        proc(eb, j, grb)

                        @pl.when(j + 4 < nb)
                        def _():
                            fire(eb, j + 4, grb, stb, sgb)

    @pl.loop(0, CHUNK)
    def _fin(r):
        for k in range(HW // 16):
            sl = pl.ds(k * 16, 16)
            a = plsc.bitcast(acc[r, sl], jnp.bfloat16)
            m = a > jnp.bfloat16(NEG_THRESH)
            sel = jnp.where(m, a, jnp.bfloat16(0.0))
            acc[r, sl] = plsc.bitcast(sel, jnp.int32)

    pltpu.async_copy(acc.at[pl.ds(0, CHUNK)], out_hbm.at[pl.ds(lo, CHUNK)],
                     sw).wait()


def kernel(x, edge_index, W1, b1, W2, b2):
    row = edge_index[0].astype(jnp.int32)
    col = edge_index[1].astype(jnp.int32)
    pad = jnp.zeros((EPAD - E,), jnp.int32)
    row_p = jnp.concatenate([row, pad])
    col_p = jnp.concatenate([col, pad])

    w1e = W1[:, 0::2]
    w1o = W1[:, 1::2]
    b1e = b1[0::2].reshape(1, HW).astype(jnp.float32)
    b1o = b1[1::2].reshape(1, HW).astype(jnp.float32)
    uv_i32 = pl.pallas_call(
        _uv_body,
        grid=(GA,),
        in_specs=[
            pl.BlockSpec((BN, D), lambda i: (lax.rem(i, GA // 2), 0)),
            pl.BlockSpec((2 * D, HW), lambda i: (0, 0)),
            pl.BlockSpec((2 * D, HW), lambda i: (0, 0)),
            pl.BlockSpec((1, HW), lambda i: (0, 0)),
            pl.BlockSpec((1, HW), lambda i: (0, 0)),
        ],
        out_specs=pl.BlockSpec((BN, HW), lambda i: (i, 0)),
        out_shape=jax.ShapeDtypeStruct((2 * N, HW), jnp.int32),
    )(x, w1e, w1o, b1e, b1o)

    mesh = plsc.VectorSubcoreMesh(
        core_axis_name="c", subcore_axis_name="s", num_cores=NC, num_subcores=NS
    )

    msg = pl.kernel(
        _gather_body,
        out_type=jax.ShapeDtypeStruct((EPAD, HW), jnp.int32),
        mesh=mesh,
        compiler_params=pltpu.CompilerParams(needs_layout_passes=False),
        scratch_types=[
            pltpu.VMEM((EPT,), jnp.int32),
            pltpu.VMEM((EPT,), jnp.int32),
            pltpu.VMEM((2 * GB,), jnp.int32),
            pltpu.VMEM((2 * GB,), jnp.int32),
            pltpu.VMEM((2 * GB,), jnp.int32),
            pltpu.VMEM((2 * GB,), jnp.int32),
            pltpu.VMEM((2 * GB, HW), jnp.int32),
            pltpu.VMEM((2 * GB, HW), jnp.int32),
            pltpu.VMEM((2 * GB, HW), jnp.int32),
            pltpu.VMEM((2 * GB, HW), jnp.int32),
            pltpu.VMEM((GB, HW), jnp.int32),
            pltpu.VMEM((GB, HW), jnp.int32),
            pltpu.SemaphoreType.DMA,
            pltpu.SemaphoreType.DMA,
            pltpu.SemaphoreType.DMA,
            pltpu.SemaphoreType.DMA,
            pltpu.SemaphoreType.DMA,
            pltpu.SemaphoreType.DMA,
        ],
    )(row_p, col_p, uv_i32)

    we = jnp.concatenate([W2[0::2, 0::2], W2[1::2, 0::2]],
                         axis=0).astype(jnp.bfloat16)
    wo = jnp.concatenate([W2[0::2, 1::2], W2[1::2, 1::2]],
                         axis=0).astype(jnp.bfloat16)
    be = b2[0::2].reshape(1, HW).astype(jnp.float32)
    bo = b2[1::2].reshape(1, HW).astype(jnp.float32)
    t = pl.pallas_call(
        _mlp2_body,
        grid=(EPAD // BE,),
        in_specs=[
            pl.BlockSpec((BE, HW), lambda i: (i, 0)),
            pl.BlockSpec((D, HW), lambda i: (0, 0)),
            pl.BlockSpec((D, HW), lambda i: (0, 0)),
            pl.BlockSpec((1, HW), lambda i: (0, 0)),
            pl.BlockSpec((1, HW), lambda i: (0, 0)),
        ],
        out_specs=pl.BlockSpec((BE, HW), lambda i: (i, 0)),
        out_shape=jax.ShapeDtypeStruct((EPAD, HW), jnp.int32),
    )(msg, we, wo, be, bo)

    out_pad = pl.kernel(
        _scatter_body,
        out_type=jax.ShapeDtypeStruct((NPAD, HW), jnp.int32),
        mesh=mesh,
        compiler_params=pltpu.CompilerParams(needs_layout_passes=False),
        scratch_types=[
            pltpu.VMEM((CD,), jnp.int32),
            pltpu.VMEM((CD,), jnp.int32),
            pltpu.VMEM((SUPER + GD,), jnp.int32),
            pltpu.VMEM((CHUNK + 1, HW), jnp.int32),
            pltpu.VMEM((GD, HW), jnp.int32),
            pltpu.VMEM((GD, HW), jnp.int32),
            pltpu.VMEM((GD, HW), jnp.int32),
            pltpu.VMEM((GD, HW), jnp.int32),
            pltpu.VMEM((GD,), jnp.int32),
            pltpu.VMEM((GD,), jnp.int32),
            pltpu.VMEM((GD,), jnp.int32),
            pltpu.VMEM((GD,), jnp.int32),
            pltpu.SemaphoreType.DMA,
            pltpu.SemaphoreType.DMA,
            pltpu.SemaphoreType.DMA,
            pltpu.SemaphoreType.DMA,
            pltpu.SemaphoreType.DMA,
            pltpu.SemaphoreType.DMA,
            pltpu.SemaphoreType.DMA,
        ],
    )(row, t)

    out_bf = jax.lax.bitcast_convert_type(out_pad, jnp.bfloat16)
    return out_bf.reshape(NPAD, D).astype(jnp.float32)[:N]
